# baseline (speedup 1.0000x reference)
import jax
import jax.numpy as jnp
from jax import lax
from jax.experimental import pallas as pl
from jax.experimental.pallas import tpu as pltpu
from jax.experimental.pallas import tpu_sc as plsc

B = 1024
N = 1024
R1 = 16384
R2 = 16384

R1_SC = 6144
R2_SC = 6144

NW = 32
BPW = B // NW
C = 6144
NG = C // 16

_P0 = 0.9997105452798941
_P1 = -0.9925717823029249
_P2 = 0.4630922557740955
_P3 = -0.10261653480185998


def _expneg(x):
    p = _P3 * x + _P2
    p = p * x + _P1
    return p * x + _P0


def _sc_body(t_hbm, y_hbm, alpha_hbm, beta_hbm,
             r1_hbm, p1_hbm, r2a_hbm, r2b_hbm, p2_hbm,
             out_hbm,
             y_v, dy_v, t_v, sc_v, a_v, b_v, i1_v, i2_v, i3_v):
    cid = lax.axis_index("c")
    sid = lax.axis_index("s")
    wid = sid * 2 + cid
    rowbase = wid * BPW

    pltpu.sync_copy(y_hbm.at[pl.ds(rowbase * N, BPW * N)], y_v)
    pltpu.sync_copy(t_hbm.at[pl.ds(rowbase, BPW)], t_v)

    for k in range(BPW // 16):
        tv = t_v[pl.ds(k * 16, 16)]
        dv = 1.0 + _expneg(tv)
        for j in range(16):
            b = k * 16 + j
            sc_v[pl.ds(b * 16, 16)] = jnp.full((16,), tv[j], jnp.float32)
            sc_v[pl.ds((BPW + b) * 16, 16)] = jnp.full((16,), dv[j],
                                                       jnp.float32)

    lane = lax.iota(jnp.int32, 16)
    shift1 = jnp.minimum(lane + 1, 15)

    def zero_body(j, carry):
        dy_v[pl.ds(j * 16, 16)] = jnp.zeros((16,), jnp.float32)
        return carry

    lax.fori_loop(0, BPW * N // 16, zero_body, 0)

    for c in range(R1_SC // C):
        off = c * C
        pltpu.sync_copy(alpha_hbm.at[pl.ds(off, C)], a_v)
        pltpu.sync_copy(beta_hbm.at[pl.ds(off, C)], b_v)
        pltpu.sync_copy(r1_hbm.at[pl.ds(off, C)], i1_v)
        pltpu.sync_copy(p1_hbm.at[pl.ds(off, C)], i2_v)

        def p1_body(g, carry):
            al = a_v[pl.ds(g * 16, 16)]
            be = b_v[pl.ds(g * 16, 16)]
            ir = i1_v[pl.ds(g * 16, 16)]
            ip = i2_v[pl.ds(g * 16, 16)]
            ipn = jnp.take(ip, shift1)
            m_int = ip != ipn
            m_end = m_int | (lane == 15)

            def row(b):
                tb = sc_v[pl.ds(b * 16, 16)]
                yrow = y_v.at[pl.ds(b * N, N)]
                dyrow = dy_v.at[pl.ds(b * N, N)]
                coeff = al * _expneg(be * tb)
                yv = plsc.load_gather(yrow, [ir])
                rate = coeff * yv
                s = plsc.cumsum(rate)
                plsc.addupdate_scatter(dyrow, [ip], s, mask=m_end)
                plsc.addupdate_scatter(dyrow, [ipn], -s, mask=m_int)
                plsc.addupdate_scatter(dyrow, [ir], -rate)

            plsc.parallel_loop(0, BPW, unroll=16)(row)
            return carry

        lax.fori_loop(0, NG, p1_body, 0)

    for c in range(R2_SC // C):
        off = R1 + c * C
        pltpu.sync_copy(alpha_hbm.at[pl.ds(off, C)], a_v)
        pltpu.sync_copy(beta_hbm.at[pl.ds(off, C)], b_v)
        pltpu.sync_copy(r2a_hbm.at[pl.ds(off - R1, C)], i1_v)
        pltpu.sync_copy(r2b_hbm.at[pl.ds(off - R1, C)], i2_v)
        pltpu.sync_copy(p2_hbm.at[pl.ds(off - R1, C)], i3_v)

        def p2_body(g, carry):
            al = a_v[pl.ds(g * 16, 16)]
            be = b_v[pl.ds(g * 16, 16)]
            ia = i1_v[pl.ds(g * 16, 16)]
            ib = i2_v[pl.ds(g * 16, 16)]
            ip = i3_v[pl.ds(g * 16, 16)]
            ipn = jnp.take(ip, shift1)
            m_int = ip != ipn
            m_end = m_int | (lane == 15)

            def row(b):
                tb = sc_v[pl.ds(b * 16, 16)]
                den = sc_v[pl.ds((BPW + b) * 16, 16)]
                yrow = y_v.at[pl.ds(b * N, N)]
                dyrow = dy_v.at[pl.ds(b * N, N)]
                coeff = al * _expneg(be * tb)
                ya = plsc.load_gather(yrow, [ia])
                yb = plsc.load_gather(yrow, [ib])
                rate = (coeff * den) * (ya * yb)
                s = plsc.cumsum(rate)
                plsc.addupdate_scatter(dyrow, [ip], s, mask=m_end)
                plsc.addupdate_scatter(dyrow, [ipn], -s, mask=m_int)
                plsc.addupdate_scatter(dyrow, [ia], -rate)
                plsc.addupdate_scatter(dyrow, [ib], -rate)

            plsc.parallel_loop(0, BPW, unroll=16)(row)
            return carry

        lax.fori_loop(0, NG, p2_body, 0)

    pltpu.sync_copy(dy_v, out_hbm.at[pl.ds(rowbase * N, BPW * N)])


def _sc_run(t_sc, y_sc_flat, alpha, beta,
            r1_idx, p1_idx, r2a_idx, r2b_idx, p2_idx):
    mesh = plsc.VectorSubcoreMesh(core_axis_name="c", subcore_axis_name="s")
    return pl.kernel(
        _sc_body,
        out_type=jax.ShapeDtypeStruct((B * N,), jnp.float32),
        mesh=mesh,
        compiler_params=pltpu.CompilerParams(needs_layout_passes=False),
        scratch_types=[
            pltpu.VMEM((BPW * N,), jnp.float32),
            pltpu.VMEM((BPW * N,), jnp.float32),
            pltpu.VMEM((BPW,), jnp.float32),
            pltpu.VMEM((2 * BPW * 16,), jnp.float32),
            pltpu.VMEM((C,), jnp.float32),
            pltpu.VMEM((C,), jnp.float32),
            pltpu.VMEM((C,), jnp.int32),
            pltpu.VMEM((C,), jnp.int32),
            pltpu.VMEM((C,), jnp.int32),
        ],
    )(t_sc, y_sc_flat, alpha, beta, r1_idx, p1_idx, r2a_idx, r2b_idx, p2_idx)


T = 1024
NSTEPS = (R1 - R1_SC) // T
TC_ROWS = B


def _tc_body(t_ref, y_ref,
             a1_ref, b1_ref, r1l_ref, r1s_ref, p1s_ref,
             a2_ref, b2_ref, r2al_ref, r2bl_ref, r2as_ref, r2bs_ref, p2s_ref,
             out_ref):
    i = pl.program_id(0)
    bf16 = jnp.bfloat16

    negt = -t_ref[...]
    den = 1.0 + jnp.exp(negt)
    y = y_ref[...]

    iota_s = lax.broadcasted_iota(jnp.int32, (N, T), 0)
    iota_l = lax.broadcasted_iota(jnp.int32, (T, N), 1)

    def dot(a, b):
        return lax.dot_general(a, b, (((1,), (0,)), ((), ())),
                               preferred_element_type=jnp.float32)

    r1l = r1l_ref[0]
    a1 = a1_ref[0]
    b1 = b1_ref[0]
    G1 = (iota_s == r1l).astype(bf16)
    g1 = dot(y, G1)
    c1 = a1 * jnp.exp(b1 * negt)
    rates1 = (c1 * g1).astype(bf16)
    r1s = r1s_ref[0]
    p1s = p1s_ref[0]
    M1 = ((p1s == iota_l).astype(jnp.float32)
          - (r1s == iota_l).astype(jnp.float32)).astype(bf16)
    acc = dot(rates1, M1)

    r2al = r2al_ref[0]
    r2bl = r2bl_ref[0]
    a2 = a2_ref[0]
    b2 = b2_ref[0]
    G2a = (iota_s == r2al).astype(bf16)
    G2b = (iota_s == r2bl).astype(bf16)
    g2a = dot(y, G2a)
    g2b = dot(y, G2b)
    c2 = a2 * jnp.exp(b2 * negt)
    rates2 = (c2 * g2a * g2b * den).astype(bf16)
    r2as = r2as_ref[0]
    r2bs = r2bs_ref[0]
    p2s = p2s_ref[0]
    M2 = ((p2s == iota_l).astype(jnp.float32)
          - (r2as == iota_l).astype(jnp.float32)
          - (r2bs == iota_l).astype(jnp.float32)).astype(bf16)
    acc = acc + dot(rates2, M2)

    @pl.when(i == 0)
    def _():
        out_ref[...] = acc

    @pl.when(i > 0)
    def _():
        out_ref[...] = out_ref[...] + acc


def _tc_run(t_col, y_bf, alpha, beta, r1_idx, p1_idx, r2a_idx, r2b_idx,
            p2_idx):
    def lanes(x):
        return x.reshape(NSTEPS, 1, T)

    def subl(x):
        return x.reshape(NSTEPS, T, 1)

    a1 = lanes(alpha[R1_SC:R1])
    a2 = lanes(alpha[R1 + R2_SC:])
    b1 = lanes(beta[R1_SC:R1])
    b2 = lanes(beta[R1 + R2_SC:])

    lane_spec = pl.BlockSpec((1, 1, T), lambda i: (i, 0, 0))
    sub_spec = pl.BlockSpec((1, T, 1), lambda i: (i, 0, 0))
    full2d = pl.BlockSpec((TC_ROWS, N), lambda i: (0, 0))

    return pl.pallas_call(
        _tc_body,
        grid=(NSTEPS,),
        in_specs=[
            pl.BlockSpec((TC_ROWS, 1), lambda i: (0, 0)),
            full2d,
            lane_spec, lane_spec,
            lane_spec, sub_spec, sub_spec,
            lane_spec, lane_spec,
            lane_spec, lane_spec,
            sub_spec, sub_spec, sub_spec,
        ],
        out_specs=full2d,
        out_shape=jax.ShapeDtypeStruct((TC_ROWS, N), jnp.float32),
        compiler_params=pltpu.CompilerParams(
            dimension_semantics=("arbitrary",),
        ),
    )(t_col, y_bf,
      a1, b1, lanes(r1_idx[R1_SC:]), subl(r1_idx[R1_SC:]),
      subl(p1_idx[R1_SC:]),
      a2, b2, lanes(r2a_idx[R2_SC:]), lanes(r2b_idx[R2_SC:]),
      subl(r2a_idx[R2_SC:]), subl(r2b_idx[R2_SC:]), subl(p2_idx[R2_SC:]))


def kernel(t_in, y_in, alpha, beta, r1_idx, p1_idx, r2a_idx, r2b_idx, p2_idx):
    out_sc = _sc_run(t_in, y_in.reshape(B * N),
                     alpha, beta, r1_idx, p1_idx, r2a_idx, r2b_idx, p2_idx)
    out_tc = _tc_run(t_in.reshape(B, 1), y_in.astype(jnp.bfloat16),
                     alpha, beta, r1_idx, p1_idx, r2a_idx, r2b_idx, p2_idx)
    return out_sc.reshape(B, N) + out_tc


# speedup vs baseline: 1.0003x; 1.0003x over previous
import jax
import jax.numpy as jnp
from jax import lax
from jax.experimental import pallas as pl
from jax.experimental.pallas import tpu as pltpu
from jax.experimental.pallas import tpu_sc as plsc

B = 1024
N = 1024
R1 = 16384
R2 = 16384

R1_SC = 6144
R2_SC = 6144

NW = 32
BPW = B // NW
C = 6144
NG = C // 16

_P0 = 0.9997105452798941
_P1 = -0.9925717823029249
_P2 = 0.4630922557740955
_P3 = -0.10261653480185998


def _expneg(x):
    p = _P3 * x + _P2
    p = p * x + _P1
    return p * x + _P0


def _sc_body(t_hbm, y_hbm, alpha_hbm, beta_hbm,
             r1_hbm, p1_hbm, r2a_hbm, r2b_hbm, p2_hbm,
             out_hbm,
             y_v, dy_v, t_v, sc_v, a_v, b_v, i1_v, i2_v, i3_v):
    cid = lax.axis_index("c")
    sid = lax.axis_index("s")
    wid = sid * 2 + cid
    rowbase = wid * BPW

    pltpu.sync_copy(y_hbm.at[pl.ds(rowbase * N, BPW * N)], y_v)
    pltpu.sync_copy(t_hbm.at[pl.ds(rowbase, BPW)], t_v)

    for k in range(BPW // 16):
        tv = t_v[pl.ds(k * 16, 16)]
        dv = 1.0 + _expneg(tv)
        for j in range(16):
            b = k * 16 + j
            sc_v[pl.ds(b * 16, 16)] = jnp.full((16,), tv[j], jnp.float32)
            sc_v[pl.ds((BPW + b) * 16, 16)] = jnp.full((16,), dv[j],
                                                       jnp.float32)

    lane = lax.iota(jnp.int32, 16)
    shift1 = jnp.minimum(lane + 1, 15)

    def zero_body(j, carry):
        dy_v[pl.ds(j * 16, 16)] = jnp.zeros((16,), jnp.float32)
        return carry

    lax.fori_loop(0, BPW * N // 16, zero_body, 0)

    for c in range(R1_SC // C):
        off = c * C
        pltpu.sync_copy(alpha_hbm.at[pl.ds(off, C)], a_v)
        pltpu.sync_copy(beta_hbm.at[pl.ds(off, C)], b_v)
        pltpu.sync_copy(r1_hbm.at[pl.ds(off, C)], i1_v)
        pltpu.sync_copy(p1_hbm.at[pl.ds(off, C)], i2_v)

        def p1_body(g, carry):
            al = a_v[pl.ds(g * 16, 16)]
            be = b_v[pl.ds(g * 16, 16)]
            ir = i1_v[pl.ds(g * 16, 16)]
            ip = i2_v[pl.ds(g * 16, 16)]
            ipn = jnp.take(ip, shift1)
            m_int = ip != ipn
            m_end = m_int | (lane == 15)

            def row(b):
                tb = sc_v[pl.ds(b * 16, 16)]
                yrow = y_v.at[pl.ds(b * N, N)]
                dyrow = dy_v.at[pl.ds(b * N, N)]
                coeff = al * _expneg(be * tb)
                yv = plsc.load_gather(yrow, [ir])
                rate = coeff * yv
                s = plsc.cumsum(rate)
                plsc.addupdate_scatter(dyrow, [ip], s, mask=m_end)
                plsc.addupdate_scatter(dyrow, [ipn], -s, mask=m_int)
                plsc.addupdate_scatter(dyrow, [ir], -rate)

            plsc.parallel_loop(0, BPW, unroll=8)(row)
            return carry

        lax.fori_loop(0, NG, p1_body, 0)

    for c in range(R2_SC // C):
        off = R1 + c * C
        pltpu.sync_copy(alpha_hbm.at[pl.ds(off, C)], a_v)
        pltpu.sync_copy(beta_hbm.at[pl.ds(off, C)], b_v)
        pltpu.sync_copy(r2a_hbm.at[pl.ds(off - R1, C)], i1_v)
        pltpu.sync_copy(r2b_hbm.at[pl.ds(off - R1, C)], i2_v)
        pltpu.sync_copy(p2_hbm.at[pl.ds(off - R1, C)], i3_v)

        def p2_body(g, carry):
            al = a_v[pl.ds(g * 16, 16)]
            be = b_v[pl.ds(g * 16, 16)]
            ia = i1_v[pl.ds(g * 16, 16)]
            ib = i2_v[pl.ds(g * 16, 16)]
            ip = i3_v[pl.ds(g * 16, 16)]
            ipn = jnp.take(ip, shift1)
            m_int = ip != ipn
            m_end = m_int | (lane == 15)

            def row(b):
                tb = sc_v[pl.ds(b * 16, 16)]
                den = sc_v[pl.ds((BPW + b) * 16, 16)]
                yrow = y_v.at[pl.ds(b * N, N)]
                dyrow = dy_v.at[pl.ds(b * N, N)]
                coeff = al * _expneg(be * tb)
                ya = plsc.load_gather(yrow, [ia])
                yb = plsc.load_gather(yrow, [ib])
                rate = (coeff * den) * (ya * yb)
                s = plsc.cumsum(rate)
                plsc.addupdate_scatter(dyrow, [ip], s, mask=m_end)
                plsc.addupdate_scatter(dyrow, [ipn], -s, mask=m_int)
                plsc.addupdate_scatter(dyrow, [ia], -rate)
                plsc.addupdate_scatter(dyrow, [ib], -rate)

            plsc.parallel_loop(0, BPW, unroll=8)(row)
            return carry

        lax.fori_loop(0, NG, p2_body, 0)

    pltpu.sync_copy(dy_v, out_hbm.at[pl.ds(rowbase * N, BPW * N)])


def _sc_run(t_sc, y_sc_flat, alpha, beta,
            r1_idx, p1_idx, r2a_idx, r2b_idx, p2_idx):
    mesh = plsc.VectorSubcoreMesh(core_axis_name="c", subcore_axis_name="s")
    return pl.kernel(
        _sc_body,
        out_type=jax.ShapeDtypeStruct((B * N,), jnp.float32),
        mesh=mesh,
        compiler_params=pltpu.CompilerParams(needs_layout_passes=False),
        scratch_types=[
            pltpu.VMEM((BPW * N,), jnp.float32),
            pltpu.VMEM((BPW * N,), jnp.float32),
            pltpu.VMEM((BPW,), jnp.float32),
            pltpu.VMEM((2 * BPW * 16,), jnp.float32),
            pltpu.VMEM((C,), jnp.float32),
            pltpu.VMEM((C,), jnp.float32),
            pltpu.VMEM((C,), jnp.int32),
            pltpu.VMEM((C,), jnp.int32),
            pltpu.VMEM((C,), jnp.int32),
        ],
    )(t_sc, y_sc_flat, alpha, beta, r1_idx, p1_idx, r2a_idx, r2b_idx, p2_idx)


T = 1024
NSTEPS = (R1 - R1_SC) // T
TC_ROWS = B


def _tc_body(t_ref, y_ref,
             a1_ref, b1_ref, r1l_ref, r1s_ref, p1s_ref,
             a2_ref, b2_ref, r2al_ref, r2bl_ref, r2as_ref, r2bs_ref, p2s_ref,
             out_ref):
    i = pl.program_id(0)
    bf16 = jnp.bfloat16

    negt = -t_ref[...]
    den = 1.0 + jnp.exp(negt)
    y = y_ref[...]

    iota_s = lax.broadcasted_iota(jnp.int32, (N, T), 0)
    iota_l = lax.broadcasted_iota(jnp.int32, (T, N), 1)

    def dot(a, b):
        return lax.dot_general(a, b, (((1,), (0,)), ((), ())),
                               preferred_element_type=jnp.float32)

    r1l = r1l_ref[0]
    a1 = a1_ref[0]
    b1 = b1_ref[0]
    G1 = (iota_s == r1l).astype(bf16)
    g1 = dot(y, G1)
    c1 = a1 * jnp.exp(b1 * negt)
    rates1 = (c1 * g1).astype(bf16)
    r1s = r1s_ref[0]
    p1s = p1s_ref[0]
    M1 = ((p1s == iota_l).astype(jnp.float32)
          - (r1s == iota_l).astype(jnp.float32)).astype(bf16)
    acc = dot(rates1, M1)

    r2al = r2al_ref[0]
    r2bl = r2bl_ref[0]
    a2 = a2_ref[0]
    b2 = b2_ref[0]
    G2a = (iota_s == r2al).astype(bf16)
    G2b = (iota_s == r2bl).astype(bf16)
    g2a = dot(y, G2a)
    g2b = dot(y, G2b)
    c2 = a2 * jnp.exp(b2 * negt)
    rates2 = (c2 * g2a * g2b * den).astype(bf16)
    r2as = r2as_ref[0]
    r2bs = r2bs_ref[0]
    p2s = p2s_ref[0]
    M2 = ((p2s == iota_l).astype(jnp.float32)
          - (r2as == iota_l).astype(jnp.float32)
          - (r2bs == iota_l).astype(jnp.float32)).astype(bf16)
    acc = acc + dot(rates2, M2)

    @pl.when(i == 0)
    def _():
        out_ref[...] = acc

    @pl.when(i > 0)
    def _():
        out_ref[...] = out_ref[...] + acc


def _tc_run(t_col, y_bf, alpha, beta, r1_idx, p1_idx, r2a_idx, r2b_idx,
            p2_idx):
    def lanes(x):
        return x.reshape(NSTEPS, 1, T)

    def subl(x):
        return x.reshape(NSTEPS, T, 1)

    a1 = lanes(alpha[R1_SC:R1])
    a2 = lanes(alpha[R1 + R2_SC:])
    b1 = lanes(beta[R1_SC:R1])
    b2 = lanes(beta[R1 + R2_SC:])

    lane_spec = pl.BlockSpec((1, 1, T), lambda i: (i, 0, 0))
    sub_spec = pl.BlockSpec((1, T, 1), lambda i: (i, 0, 0))
    full2d = pl.BlockSpec((TC_ROWS, N), lambda i: (0, 0))

    return pl.pallas_call(
        _tc_body,
        grid=(NSTEPS,),
        in_specs=[
            pl.BlockSpec((TC_ROWS, 1), lambda i: (0, 0)),
            full2d,
            lane_spec, lane_spec,
            lane_spec, sub_spec, sub_spec,
            lane_spec, lane_spec,
            lane_spec, lane_spec,
            sub_spec, sub_spec, sub_spec,
        ],
        out_specs=full2d,
        out_shape=jax.ShapeDtypeStruct((TC_ROWS, N), jnp.float32),
        compiler_params=pltpu.CompilerParams(
            dimension_semantics=("arbitrary",),
        ),
    )(t_col, y_bf,
      a1, b1, lanes(r1_idx[R1_SC:]), subl(r1_idx[R1_SC:]),
      subl(p1_idx[R1_SC:]),
      a2, b2, lanes(r2a_idx[R2_SC:]), lanes(r2b_idx[R2_SC:]),
      subl(r2a_idx[R2_SC:]), subl(r2b_idx[R2_SC:]), subl(p2_idx[R2_SC:]))


def kernel(t_in, y_in, alpha, beta, r1_idx, p1_idx, r2a_idx, r2b_idx, p2_idx):
    out_sc = _sc_run(t_in, y_in.reshape(B * N),
                     alpha, beta, r1_idx, p1_idx, r2a_idx, r2b_idx, p2_idx)
    out_tc = _tc_run(t_in.reshape(B, 1), y_in.astype(jnp.bfloat16),
                     alpha, beta, r1_idx, p1_idx, r2a_idx, r2b_idx, p2_idx)
    return out_sc.reshape(B, N) + out_tc
